# baseline (speedup 1.0000x reference)
import functools

import jax
import jax.numpy as jnp
from jax import lax
from jax.experimental import pallas as pl
from jax.experimental.pallas import tpu as pltpu
from jax.experimental.pallas import tpu_sc as plsc

_INFO = plsc.get_sparse_core_info()
_NC = _INFO.num_cores
_NS = _INFO.num_subcores
_NW = _NC * _NS

_B = 16384
_H = 200
_D = 32
_BPW = _B // _NW
_BT = _BPW // 128


def _body(x_hbm, w_hbm, out_hbm,
          idx0, idx1, rv0, rv1, tv0, tv1, iv,
          s_i0, s_i1, s_g0, s_g1, s_s0, s_s1):
  wid = lax.axis_index("s") * _NC + lax.axis_index("c")
  b0 = wid * _BPW
  bt0 = wid * _BT
  idx = (idx0, idx1)
  rv = (rv0, rv1)
  tv = (tv0, tv1)
  s_i = (s_i0, s_i1)
  s_g = (s_g0, s_g1)
  s_s = (s_s0, s_s1)
  iota = lax.iota(jnp.int32, 16)

  def x_slice(h):
    return x_hbm.at[pl.ds(h * _B + b0, _BPW)]

  def issue_stores(h, b):
    for ct in range(4):
      pltpu.async_copy(tv[b].at[ct], out_hbm.at[h, ct, pl.ds(bt0, _BT)],
                       s_s[b])

  def drain_stores(h, b):
    for ct in range(4):
      pltpu.make_async_copy(tv[b].at[ct], out_hbm.at[h, ct, pl.ds(bt0, _BT)],
                            s_s[b]).wait()

  def shuffle(b):
    @plsc.parallel_loop(0, _BPW, unroll=8)
    def _(r):
      rbc = jnp.full((16,), r, jnp.int32)
      plsc.store_scatter(iv, [iota, rbc], rv[b][r, pl.ds(0, 16)])
      plsc.store_scatter(iv, [iota + 16, rbc], rv[b][r, pl.ds(16, 16)])

    @plsc.parallel_loop(0, _BT * 8, unroll=4)
    def _(n):
      btl = n // 8
      b1g = n % 8
      off = btl * 128 + b1g * 16
      for ct in range(4):
        for c8 in range(8):
          tv[b][ct, btl, c8, pl.ds(b1g * 16, 16)] = (
              iv[ct * 8 + c8, pl.ds(off, 16)])

  pltpu.async_copy(x_slice(0), idx[0], s_i[0])
  pltpu.async_copy(x_slice(1), idx[1], s_i[1])
  pltpu.make_async_copy(x_slice(0), idx[0], s_i[0]).wait()
  pltpu.async_copy(w_hbm.at[idx[0]], rv[0], s_g[0])

  def pair(g, carry):
    for b in range(2):
      h = g * 2 + b

      @pl.when(g > 0)
      def _():
        drain_stores(h, b)

      pltpu.make_async_copy(w_hbm.at[idx[b]], rv[b], s_g[b]).wait()
      pltpu.async_copy(x_slice(lax.rem(h + 2, _H)), idx[b], s_i[b])
      pltpu.make_async_copy(x_slice(h), idx[1 - b], s_i[1 - b]).wait()
      if b == 0:
        pltpu.async_copy(w_hbm.at[idx[1 - b]], rv[1 - b], s_g[1 - b])
      else:
        @pl.when(g < _H // 2 - 1)
        def _():
          pltpu.async_copy(w_hbm.at[idx[1 - b]], rv[1 - b], s_g[1 - b])
      shuffle(b)
      issue_stores(h, b)
    return carry

  lax.fori_loop(0, _H // 2, pair, 0)

  drain_stores(_H - 2, 0)
  drain_stores(_H - 1, 1)
  pltpu.make_async_copy(x_slice(1), idx[1], s_i[1]).wait()


_V = 1000000
_TPW = _V // _NW
_CH = 512


def _wt_body(wt_hbm, wrow_hbm, wv0, wv1, ov0, ov1, pv,
             s_i0, s_i1, s_o0, s_o1):
  wid = lax.axis_index("s") * _NC + lax.axis_index("c")
  iota = lax.iota(jnp.int32, 16)
  wv = (wv0, wv1)
  ov = (ov0, ov1)
  s_i = (s_i0, s_i1)
  s_o = (s_o0, s_o1)
  s = (wid * _TPW) // 16 * 16
  e = ((wid + 1) * _TPW) // 16 * 16
  n_chunks = 62

  def off_of(k):
    return lax.min(s + k * _CH, e - _CH)

  def in_copy(k, b):
    return pltpu.make_async_copy(
        wt_hbm.at[:, pl.ds(off_of(k), _CH)], wv[b], s_i[b])

  def out_copy(k, b):
    return pltpu.make_async_copy(
        ov[b], wrow_hbm.at[pl.ds(off_of(k), _CH)], s_o[b])

  def shuffle(b):
    @plsc.parallel_loop(0, _CH // 16, unroll=4)
    def _(g):
      rowv = iota + g * 16
      for c in range(_D):
        plsc.store_scatter(pv, [rowv, jnp.full((16,), c, jnp.int32)],
                           wv[b][c, pl.ds(g * 16, 16)])

    @plsc.parallel_loop(0, _CH, unroll=8)
    def _(t):
      ov[b][t, pl.ds(0, 16)] = pv[t, pl.ds(0, 16)]
      ov[b][t, pl.ds(16, 16)] = pv[t, pl.ds(16, 16)]

  in_copy(0, 0).start()

  def pair(g, carry):
    for b in range(2):
      k = g * 2 + b
      in_copy(k, b).wait()

      @pl.when(k < n_chunks - 1)
      def _():
        in_copy(k + 1, 1 - b).start()

      @pl.when(g > 0)
      def _():
        out_copy(k, b).wait()

      shuffle(b)
      out_copy(k, b).start()
    return carry

  lax.fori_loop(0, n_chunks // 2, pair, 0)
  out_copy(n_chunks - 2, 0).wait()
  out_copy(n_chunks - 1, 1).wait()


@jax.jit
def _transpose_w(wt):
  mesh = plsc.VectorSubcoreMesh(core_axis_name="c", subcore_axis_name="s")
  return pl.kernel(
      _wt_body,
      out_type=jax.ShapeDtypeStruct((_V, _D), jnp.float32),
      mesh=mesh,
      compiler_params=pltpu.CompilerParams(use_tc_tiling_on_sc=False,
                                           needs_layout_passes=False),
      scratch_types=[
          pltpu.VMEM((_D, _CH), jnp.float32),
          pltpu.VMEM((_D, _CH), jnp.float32),
          pltpu.VMEM((_CH, _D), jnp.float32),
          pltpu.VMEM((_CH, _D), jnp.float32),
          pltpu.VMEM((_CH, 33), jnp.float32),
          pltpu.SemaphoreType.DMA,
          pltpu.SemaphoreType.DMA,
          pltpu.SemaphoreType.DMA,
          pltpu.SemaphoreType.DMA,
      ],
  )(wt)


@jax.jit
def _gather(xt_flat, w):
  mesh = plsc.VectorSubcoreMesh(core_axis_name="c", subcore_axis_name="s")
  return pl.kernel(
      _body,
      out_type=jax.ShapeDtypeStruct((_H, _D // 8, _B // 128, 8, 128),
                                    jnp.float32),
      mesh=mesh,
      compiler_params=pltpu.CompilerParams(use_tc_tiling_on_sc=False,
                                           needs_layout_passes=False),
      scratch_types=[
          pltpu.VMEM((_BPW,), jnp.int32),
          pltpu.VMEM((_BPW,), jnp.int32),
          pltpu.VMEM((_BPW, _D), jnp.float32),
          pltpu.VMEM((_BPW, _D), jnp.float32),
          pltpu.VMEM((_D // 8, _BT, 8, 128), jnp.float32),
          pltpu.VMEM((_D // 8, _BT, 8, 128), jnp.float32),
          pltpu.VMEM((_D, 513), jnp.float32),
          pltpu.SemaphoreType.DMA,
          pltpu.SemaphoreType.DMA,
          pltpu.SemaphoreType.DMA,
          pltpu.SemaphoreType.DMA,
          pltpu.SemaphoreType.DMA,
          pltpu.SemaphoreType.DMA,
      ],
  )(xt_flat, w)


def kernel(x, W):
  b, h = x.shape
  _, d = W.shape
  xt_flat = x.T.reshape(b * h).astype(jnp.int32)
  w_row = _transpose_w(W.T)
  o5 = _gather(xt_flat, w_row)
  return o5.transpose(2, 4, 0, 1, 3).reshape(b, h, d)


# speedup vs baseline: 3.4747x; 3.4747x over previous
import functools

import jax
import jax.numpy as jnp
from jax import lax
from jax.experimental import pallas as pl
from jax.experimental.pallas import tpu as pltpu
from jax.experimental.pallas import tpu_sc as plsc

_INFO = plsc.get_sparse_core_info()
_NC = _INFO.num_cores
_NS = _INFO.num_subcores
_NW = _NC * _NS

_B = 16384
_H = 200
_D = 32
_BPW = _B // _NW
_BT = _BPW // 128


def _body(x_hbm, w_hbm, out_hbm,
          idx0, idx1, rv0, rv1, tv0, tv1, iv,
          s_i0, s_i1, s_g0, s_g1, s_s0, s_s1):
  wid = lax.axis_index("s") * _NC + lax.axis_index("c")
  b0 = wid * _BPW
  bt0 = wid * _BT
  idx = (idx0, idx1)
  rv = (rv0, rv1)
  tv = (tv0, tv1)
  s_i = (s_i0, s_i1)
  s_g = (s_g0, s_g1)
  s_s = (s_s0, s_s1)
  iota = lax.iota(jnp.int32, 16)

  def x_slice(h):
    return x_hbm.at[pl.ds(h * _B + b0, _BPW)]

  def issue_stores(h, b):
    for ct in range(4):
      pltpu.async_copy(tv[b].at[ct], out_hbm.at[h, ct, pl.ds(bt0, _BT)],
                       s_s[b])

  def drain_stores(h, b):
    for ct in range(4):
      pltpu.make_async_copy(tv[b].at[ct], out_hbm.at[h, ct, pl.ds(bt0, _BT)],
                            s_s[b]).wait()

  def shuffle(b):
    cv_lo = iota * 513
    cv_hi = (iota + 16) * 513

    @plsc.parallel_loop(0, _BPW, unroll=8)
    def _(r):
      plsc.store_scatter(iv, [cv_lo + r], rv[b][r, pl.ds(0, 16)])
      plsc.store_scatter(iv, [cv_hi + r], rv[b][r, pl.ds(16, 16)])

    @plsc.parallel_loop(0, _BT * 8, unroll=8)
    def _(n):
      btl = n // 8
      b1g = n % 8
      off = btl * 128 + b1g * 16
      for ct in range(4):
        for c8 in range(8):
          tv[b][ct, btl, c8, pl.ds(b1g * 16, 16)] = (
              iv[pl.ds((ct * 8 + c8) * 513 + off, 16)])

  pltpu.async_copy(x_slice(0), idx[0], s_i[0])
  pltpu.async_copy(x_slice(1), idx[1], s_i[1])
  pltpu.make_async_copy(x_slice(0), idx[0], s_i[0]).wait()
  pltpu.async_copy(w_hbm.at[idx[0]], rv[0], s_g[0])

  def pair(g, carry):
    for b in range(2):
      h = g * 2 + b

      @pl.when(g > 0)
      def _():
        drain_stores(h, b)

      pltpu.make_async_copy(w_hbm.at[idx[b]], rv[b], s_g[b]).wait()
      pltpu.async_copy(x_slice(lax.rem(h + 2, _H)), idx[b], s_i[b])
      pltpu.make_async_copy(x_slice(h), idx[1 - b], s_i[1 - b]).wait()
      if b == 0:
        pltpu.async_copy(w_hbm.at[idx[1 - b]], rv[1 - b], s_g[1 - b])
      else:
        @pl.when(g < _H // 2 - 1)
        def _():
          pltpu.async_copy(w_hbm.at[idx[1 - b]], rv[1 - b], s_g[1 - b])
      shuffle(b)
      issue_stores(h, b)
    return carry

  lax.fori_loop(0, _H // 2, pair, 0)

  drain_stores(_H - 2, 0)
  drain_stores(_H - 1, 1)
  pltpu.make_async_copy(x_slice(1), idx[1], s_i[1]).wait()


@jax.jit
def _gather(xt_flat, w):
  mesh = plsc.VectorSubcoreMesh(core_axis_name="c", subcore_axis_name="s")
  return pl.kernel(
      _body,
      out_type=jax.ShapeDtypeStruct((_H, _D // 8, _B // 128, 8, 128),
                                    jnp.float32),
      mesh=mesh,
      compiler_params=pltpu.CompilerParams(use_tc_tiling_on_sc=False,
                                           needs_layout_passes=False),
      scratch_types=[
          pltpu.VMEM((_BPW,), jnp.int32),
          pltpu.VMEM((_BPW,), jnp.int32),
          pltpu.VMEM((_BPW, _D), jnp.float32),
          pltpu.VMEM((_BPW, _D), jnp.float32),
          pltpu.VMEM((_D // 8, _BT, 8, 128), jnp.float32),
          pltpu.VMEM((_D // 8, _BT, 8, 128), jnp.float32),
          pltpu.VMEM((_D * 513,), jnp.float32),
          pltpu.SemaphoreType.DMA,
          pltpu.SemaphoreType.DMA,
          pltpu.SemaphoreType.DMA,
          pltpu.SemaphoreType.DMA,
          pltpu.SemaphoreType.DMA,
          pltpu.SemaphoreType.DMA,
      ],
  )(xt_flat, w)


def kernel(x, W):
  b, h = x.shape
  _, d = W.shape
  xt_flat = x.T.reshape(b * h).astype(jnp.int32)
  o5 = _gather(xt_flat, W)
  return o5.transpose(2, 4, 0, 1, 3).reshape(b, h, d)


# speedup vs baseline: 3.5448x; 1.0202x over previous
import functools

import jax
import jax.numpy as jnp
from jax import lax
from jax.experimental import pallas as pl
from jax.experimental.pallas import tpu as pltpu
from jax.experimental.pallas import tpu_sc as plsc

_INFO = plsc.get_sparse_core_info()
_NC = _INFO.num_cores
_NS = _INFO.num_subcores
_NW = _NC * _NS

_B = 16384
_H = 200
_D = 32
_BPW = _B // _NW
_BT = _BPW // 128


def _body(x_hbm, w_hbm, out_hbm,
          idx0, idx1, rv0, rv1, tv0, tv1, iv,
          s_i0, s_i1, s_g0, s_g1, s_s0, s_s1):
  wid = lax.axis_index("s") * _NC + lax.axis_index("c")
  b0 = wid * _BPW
  bt0 = wid * _BT
  idx = (idx0, idx1)
  rv = (rv0, rv1)
  tv = (tv0, tv1)
  s_i = (s_i0, s_i1)
  s_g = (s_g0, s_g1)
  s_s = (s_s0, s_s1)
  iota = lax.iota(jnp.int32, 16)

  def x_slice(h):
    return x_hbm.at[pl.ds(h * _B + b0, _BPW)]

  def issue_stores(h, b):
    for ct in range(4):
      pltpu.async_copy(tv[b].at[ct], out_hbm.at[h, ct, pl.ds(bt0, _BT)],
                       s_s[b])

  def drain_stores(h, b):
    for ct in range(4):
      pltpu.make_async_copy(tv[b].at[ct], out_hbm.at[h, ct, pl.ds(bt0, _BT)],
                            s_s[b]).wait()

  def shuffle(b):
    @plsc.parallel_loop(0, _BPW, unroll=16)
    def _(r):
      rbc = jnp.full((16,), r, jnp.int32)
      plsc.store_scatter(iv, [iota, rbc], rv[b][r, pl.ds(0, 16)])
      plsc.store_scatter(iv, [iota + 16, rbc], rv[b][r, pl.ds(16, 16)])

    @plsc.parallel_loop(0, _BT * 8, unroll=4)
    def _(n):
      btl = n // 8
      b1g = n % 8
      off = btl * 128 + b1g * 16
      for ct in range(4):
        for c8 in range(8):
          tv[b][ct, btl, c8, pl.ds(b1g * 16, 16)] = (
              iv[ct * 8 + c8, pl.ds(off, 16)])

  pltpu.async_copy(x_slice(0), idx[0], s_i[0])
  pltpu.async_copy(x_slice(1), idx[1], s_i[1])
  pltpu.make_async_copy(x_slice(0), idx[0], s_i[0]).wait()
  pltpu.async_copy(w_hbm.at[idx[0]], rv[0], s_g[0])

  def pair(g, carry):
    for b in range(2):
      h = g * 2 + b

      @pl.when(g > 0)
      def _():
        drain_stores(h, b)

      pltpu.make_async_copy(w_hbm.at[idx[b]], rv[b], s_g[b]).wait()
      pltpu.async_copy(x_slice(lax.rem(h + 2, _H)), idx[b], s_i[b])
      pltpu.make_async_copy(x_slice(h), idx[1 - b], s_i[1 - b]).wait()
      if b == 0:
        pltpu.async_copy(w_hbm.at[idx[1 - b]], rv[1 - b], s_g[1 - b])
      else:
        @pl.when(g < _H // 2 - 1)
        def _():
          pltpu.async_copy(w_hbm.at[idx[1 - b]], rv[1 - b], s_g[1 - b])
      shuffle(b)
      issue_stores(h, b)
    return carry

  lax.fori_loop(0, _H // 2, pair, 0)

  drain_stores(_H - 2, 0)
  drain_stores(_H - 1, 1)
  pltpu.make_async_copy(x_slice(1), idx[1], s_i[1]).wait()


@jax.jit
def _gather(xt_flat, w):
  mesh = plsc.VectorSubcoreMesh(core_axis_name="c", subcore_axis_name="s")
  return pl.kernel(
      _body,
      out_type=jax.ShapeDtypeStruct((_H, _D // 8, _B // 128, 8, 128),
                                    jnp.float32),
      mesh=mesh,
      compiler_params=pltpu.CompilerParams(use_tc_tiling_on_sc=False,
                                           needs_layout_passes=False),
      scratch_types=[
          pltpu.VMEM((_BPW,), jnp.int32),
          pltpu.VMEM((_BPW,), jnp.int32),
          pltpu.VMEM((_BPW, _D), jnp.float32),
          pltpu.VMEM((_BPW, _D), jnp.float32),
          pltpu.VMEM((_D // 8, _BT, 8, 128), jnp.float32),
          pltpu.VMEM((_D // 8, _BT, 8, 128), jnp.float32),
          pltpu.VMEM((_D, 513), jnp.float32),
          pltpu.SemaphoreType.DMA,
          pltpu.SemaphoreType.DMA,
          pltpu.SemaphoreType.DMA,
          pltpu.SemaphoreType.DMA,
          pltpu.SemaphoreType.DMA,
          pltpu.SemaphoreType.DMA,
      ],
  )(xt_flat, w)


def kernel(x, W):
  b, h = x.shape
  _, d = W.shape
  xt_flat = x.T.reshape(b * h).astype(jnp.int32)
  o5 = _gather(xt_flat, W)
  return o5.transpose(2, 4, 0, 1, 3).reshape(b, h, d)
